# baseline (speedup 1.0000x reference)
import functools

import jax
import jax.numpy as jnp
import numpy as np
from jax import lax
from jax.experimental import pallas as pl
from jax.experimental.pallas import tpu as pltpu
from jax.experimental.pallas import tpu_sc as plsc

D = 1024
F = 4096
E = 8
T = 2048
LANES = 128
BM = 256
NBLK = (T * 2) // BM + E
TBLK = 256
N_PAD = NBLK * BM

NC, NS = 2, 16
NW = NC * NS
TPW = T // NW
CHUNK = 32

_NEG = np.float32(-1e30)


def _top2(logits):
    lane = lax.broadcasted_iota(jnp.int32, logits.shape, 1)
    m1 = jnp.max(logits, axis=1, keepdims=True)
    i1 = jnp.min(jnp.where(logits == m1, lane, LANES), axis=1, keepdims=True)
    oh1 = (lane == i1).astype(jnp.float32)
    l2 = jnp.where(lane == i1, _NEG, logits)
    m2 = jnp.max(l2, axis=1, keepdims=True)
    i2 = jnp.min(jnp.where(l2 == m2, lane, LANES), axis=1, keepdims=True)
    oh2 = (lane == i2).astype(jnp.float32)
    return oh1, oh2, m1, m2


def _router_body(logits_ref, lt_ref, mtri_ref,
                 pos0_ref, pos1_ref, g0_ref, g1_ref, counts_ref,
                 counts_acc, run_acc, off_acc):
    p = pl.program_id(0)
    i = pl.program_id(1)
    logits = logits_ref[...]
    oh1, oh2, m1, m2 = _top2(logits)
    s = oh1 + oh2

    @pl.when(jnp.logical_and(p == 0, i == 0))
    def _init():
        counts_acc[...] = jnp.zeros_like(counts_acc)

    @pl.when(p == 0)
    def _count():
        counts_acc[...] += jnp.sum(s, axis=0, keepdims=True)
        counts_ref[...] = jnp.broadcast_to(counts_acc[...], counts_ref.shape)

    @pl.when(jnp.logical_and(p == 1, i == 0))
    def _offsets():
        c = counts_acc[...]
        padded = jnp.floor((c + (BM - 1)) / BM) * BM
        off_acc[...] = jnp.dot(padded, mtri_ref[...],
                               preferred_element_type=jnp.float32)
        run_acc[...] = jnp.zeros_like(run_acc)

    @pl.when(p == 1)
    def _emit():
        c_blk = jnp.dot(lt_ref[...], s, preferred_element_type=jnp.float32)
        tot = c_blk + run_acc[...] + off_acc[...]
        run_acc[...] += jnp.sum(s, axis=0, keepdims=True)
        pos0 = jnp.sum(tot * oh1, axis=1, keepdims=True).astype(jnp.int32)
        pos1 = jnp.sum(tot * oh2, axis=1, keepdims=True).astype(jnp.int32)
        pos0_ref[...] = jnp.broadcast_to(pos0, pos0_ref.shape)
        pos1_ref[...] = jnp.broadcast_to(pos1, pos1_ref.shape)
        g1v = 1.0 / (1.0 + jnp.exp(m2 - m1))
        g0_ref[...] = jnp.broadcast_to(g1v, g0_ref.shape)
        g1_ref[...] = jnp.broadcast_to(1.0 - g1v, g1_ref.shape)


def _run_router(logits_pad, lt, mtri):
    nb = T // TBLK

    def tok_spec(dtype):
        return (pl.BlockSpec((TBLK, LANES),
                             lambda p, i: (jnp.where(p == 1, i, nb), 0)),
                jax.ShapeDtypeStruct((T + TBLK, LANES), dtype))

    specs = [tok_spec(jnp.int32), tok_spec(jnp.int32),
             tok_spec(jnp.float32), tok_spec(jnp.float32)]
    return pl.pallas_call(
        _router_body,
        grid=(2, nb),
        in_specs=[
            pl.BlockSpec((TBLK, LANES), lambda p, i: (i, 0)),
            pl.BlockSpec((TBLK, TBLK), lambda p, i: (0, 0)),
            pl.BlockSpec((LANES, LANES), lambda p, i: (0, 0)),
        ],
        out_specs=[s for s, _ in specs] + [
            pl.BlockSpec((8, LANES), lambda p, i: (0, 0))],
        out_shape=[o for _, o in specs] + [
            jax.ShapeDtypeStruct((8, LANES), jnp.float32)],
        scratch_shapes=[
            pltpu.VMEM((1, LANES), jnp.float32),
            pltpu.VMEM((1, LANES), jnp.float32),
            pltpu.VMEM((1, LANES), jnp.float32),
        ],
    )(logits_pad, lt, mtri)


@functools.lru_cache(maxsize=None)
def _make_dispatch():
    mesh = plsc.VectorSubcoreMesh(core_axis_name="c", subcore_axis_name="s",
                                  num_cores=NC, num_subcores=NS)

    @functools.partial(
        pl.kernel,
        mesh=mesh,
        out_type=[
            jax.ShapeDtypeStruct((N_PAD, D), jnp.float32),
            jax.ShapeDtypeStruct((N_PAD, LANES), jnp.float32),
        ],
        scratch_types=[
            pltpu.VMEM((TPW, D), jnp.float32),
            pltpu.VMEM((TPW, LANES), jnp.float32),
            pltpu.VMEM((TPW, LANES), jnp.float32),
            pltpu.VMEM((TPW,), jnp.int32),
            pltpu.VMEM((TPW,), jnp.int32),
            pltpu.SemaphoreType.DMA,
        ],
    )
    def _dispatch(x_hbm, pos0_hbm, pos1_hbm, g0_hbm, g1_hbm,
                  xpad_hbm, gsort_hbm,
                  rows_v, g0_v, g1_v, idx0_v, idx1_v, sem):
        wid = lax.axis_index("s") * NC + lax.axis_index("c")
        base = wid * TPW
        pltpu.sync_copy(x_hbm.at[pl.ds(base, TPW)], rows_v)
        pltpu.sync_copy(pos0_hbm.at[pl.ds(base, TPW)], idx0_v)
        pltpu.sync_copy(pos1_hbm.at[pl.ds(base, TPW)], idx1_v)
        pltpu.sync_copy(g0_hbm.at[pl.ds(base, TPW)], g0_v)
        pltpu.sync_copy(g1_hbm.at[pl.ds(base, TPW)], g1_v)
        c0 = pltpu.async_copy(rows_v, xpad_hbm.at[idx0_v], sem)
        c1 = pltpu.async_copy(rows_v, xpad_hbm.at[idx1_v], sem)
        c2 = pltpu.async_copy(g0_v, gsort_hbm.at[idx0_v], sem)
        c3 = pltpu.async_copy(g1_v, gsort_hbm.at[idx1_v], sem)
        c0.wait()
        c1.wait()
        c2.wait()
        c3.wait()

    return _dispatch


FH = F // 2


def _gmm_body(be_ref, bv_ref, x_ref, g_ref, w1_ref, w2_ref, o_ref, acc_ref):
    f = pl.program_id(0)
    m = pl.program_id(1)

    @pl.when(bv_ref[m] == 1)
    def _():
        x = x_ref[...].astype(jnp.bfloat16)
        h = jnp.dot(x, w1_ref[0].astype(jnp.bfloat16),
                    preferred_element_type=jnp.float32)
        h = jax.nn.gelu(h)
        part = jnp.dot(h.astype(jnp.bfloat16), w2_ref[0].astype(jnp.bfloat16),
                       preferred_element_type=jnp.float32)
        sl = pl.ds(pl.multiple_of(m * BM, BM), BM)

        @pl.when(f == 0)
        def _store():
            acc_ref[sl, :] = part.astype(jnp.bfloat16)

        @pl.when(f == 1)
        def _emit():
            o_ref[...] = (acc_ref[sl, :].astype(jnp.float32) + part) \
                * g_ref[:, 0:1]


def _run_gmm(block_expert, block_valid, xpad, gsort, w1, w2):
    spec = pltpu.PrefetchScalarGridSpec(
        num_scalar_prefetch=2,
        grid=(2, NBLK),
        in_specs=[
            pl.BlockSpec((BM, D), lambda f, m, be, bv: (m, 0)),
            pl.BlockSpec((BM, LANES),
                         lambda f, m, be, bv: (jnp.where(f == 1, m, 0), 0)),
            pl.BlockSpec((1, D, FH), lambda f, m, be, bv: (be[m], 0, f)),
            pl.BlockSpec((1, FH, D), lambda f, m, be, bv: (be[m], f, 0)),
        ],
        out_specs=pl.BlockSpec(
            (BM, D),
            lambda f, m, be, bv: (jnp.where(f == 1, m, NBLK), 0)),
        scratch_shapes=[pltpu.VMEM((N_PAD, D), jnp.bfloat16)],
    )
    out = pl.pallas_call(
        _gmm_body,
        grid_spec=spec,
        out_shape=jax.ShapeDtypeStruct(((NBLK + 1) * BM, D), jnp.float32),
    )(block_expert, block_valid, xpad, gsort, w1, w2)
    return out


def _shared_body(x_ref, w1_ref, w2_ref, o_ref, acc_ref):
    f = pl.program_id(0)
    i = pl.program_id(1)
    x = x_ref[...].astype(jnp.bfloat16)
    h = jax.nn.gelu(jnp.dot(x, w1_ref[...].astype(jnp.bfloat16),
                            preferred_element_type=jnp.float32))
    part = jnp.dot(h.astype(jnp.bfloat16), w2_ref[...].astype(jnp.bfloat16),
                   preferred_element_type=jnp.float32)
    sl = pl.ds(pl.multiple_of(i * TBLK, TBLK), TBLK)

    @pl.when(f == 0)
    def _store():
        acc_ref[sl, :] = part.astype(jnp.bfloat16)

    @pl.when(f == 1)
    def _emit():
        o_ref[...] = acc_ref[sl, :].astype(jnp.float32) + part


def _run_shared(x, w1, w2):
    nb = T // TBLK
    return pl.pallas_call(
        _shared_body,
        grid=(2, nb),
        in_specs=[
            pl.BlockSpec((TBLK, D), lambda f, i: (i, 0)),
            pl.BlockSpec((D, FH), lambda f, i: (0, f)),
            pl.BlockSpec((FH, D), lambda f, i: (f, 0)),
        ],
        out_specs=pl.BlockSpec(
            (TBLK, D), lambda f, i: (jnp.where(f == 1, i, nb), 0)),
        out_shape=jax.ShapeDtypeStruct((T + TBLK, D), jnp.float32),
        scratch_shapes=[pltpu.VMEM((T, D), jnp.bfloat16)],
    )(x, w1, w2)


@functools.lru_cache(maxsize=None)
def _make_combine():
    mesh = plsc.VectorSubcoreMesh(core_axis_name="c", subcore_axis_name="s",
                                  num_cores=NC, num_subcores=NS)

    @functools.partial(
        pl.kernel,
        mesh=mesh,
        out_type=jax.ShapeDtypeStruct((T, D), jnp.float32),
        scratch_types=[
            pltpu.VMEM((CHUNK, D), jnp.float32),
            pltpu.VMEM((CHUNK, D), jnp.float32),
            pltpu.VMEM((CHUNK, D), jnp.float32),
            pltpu.VMEM((CHUNK,), jnp.int32),
            pltpu.VMEM((CHUNK,), jnp.int32),
            pltpu.SemaphoreType.DMA,
        ],
    )
    def _combine(shared_hbm, osort_hbm, pos0_hbm, pos1_hbm, out_hbm,
                 r0_v, r1_v, sh_v, idx0_v, idx1_v, sem):
        wid = lax.axis_index("s") * NC + lax.axis_index("c")
        for chunk in range(TPW // CHUNK):
            base = wid * TPW + chunk * CHUNK
            pltpu.sync_copy(pos0_hbm.at[pl.ds(base, CHUNK)], idx0_v)
            pltpu.sync_copy(pos1_hbm.at[pl.ds(base, CHUNK)], idx1_v)
            a = pltpu.async_copy(osort_hbm.at[idx0_v], r0_v, sem)
            b = pltpu.async_copy(osort_hbm.at[idx1_v], r1_v, sem)
            pltpu.sync_copy(shared_hbm.at[pl.ds(base, CHUNK)], sh_v)
            a.wait()
            b.wait()

            def col(j, _):
                sl = pl.ds(pl.multiple_of(j * 16, 16), 16)
                for i in range(CHUNK):
                    plsc.addupdate(sh_v.at[i, sl], r0_v[i, sl] + r1_v[i, sl])
                return 0

            lax.fori_loop(0, D // 16, col, 0)
            pltpu.sync_copy(sh_v, out_hbm.at[pl.ds(base, CHUNK)])

    return _combine


_LT = np.tril(np.ones((TBLK, TBLK), np.float32), k=-1)
_MTRI = np.triu(np.ones((LANES, LANES), np.float32), k=1)


def kernel(hidden_states, router_w, shared_w1, shared_w2, expert_w1, expert_w2):
    logits = hidden_states @ router_w
    logits_pad = jnp.pad(logits, ((0, 0), (0, LANES - E)),
                         constant_values=-1e30)

    lt = jnp.asarray(_LT)
    mtri = jnp.asarray(_MTRI)
    pos0w, pos1w, g0w, g1w, counts_w = _run_router(logits_pad, lt, mtri)
    pos0 = pos0w[:T, 0]
    pos1 = pos1w[:T, 0]

    counts = counts_w[0, :E].astype(jnp.int32)
    padded = ((counts + BM - 1) // BM) * BM
    incl = jnp.cumsum(padded)
    starts = jnp.arange(NBLK, dtype=jnp.int32) * BM
    block_expert = jnp.minimum(
        jnp.searchsorted(incl, starts, side="right").astype(jnp.int32), E - 1)
    block_valid = (starts < incl[E - 1]).astype(jnp.int32)

    xpad, gsort = _make_dispatch()(hidden_states, pos0, pos1, g0w, g1w)

    osort = _run_gmm(block_expert, block_valid, xpad, gsort,
                     expert_w1, expert_w2)

    shared = _run_shared(hidden_states, shared_w1, shared_w2)

    return _make_combine()(shared, osort, pos0, pos1)


# speedup vs baseline: 1.0301x; 1.0301x over previous
import functools

import jax
import jax.numpy as jnp
import numpy as np
from jax import lax
from jax.experimental import pallas as pl
from jax.experimental.pallas import tpu as pltpu
from jax.experimental.pallas import tpu_sc as plsc

D = 1024
F = 4096
E = 8
T = 2048
LANES = 128
BM = 256
NBLK = (T * 2) // BM + E
TBLK = 256
N_PAD = NBLK * BM

NC, NS = 2, 16
NW = NC * NS
TPW = T // NW
CHUNK = 32

_NEG = np.float32(-1e30)


def _top2(logits):
    lane = lax.broadcasted_iota(jnp.int32, logits.shape, 1)
    m1 = jnp.max(logits, axis=1, keepdims=True)
    i1 = jnp.min(jnp.where(logits == m1, lane, LANES), axis=1, keepdims=True)
    oh1 = (lane == i1).astype(jnp.float32)
    l2 = jnp.where(lane == i1, _NEG, logits)
    m2 = jnp.max(l2, axis=1, keepdims=True)
    i2 = jnp.min(jnp.where(l2 == m2, lane, LANES), axis=1, keepdims=True)
    oh2 = (lane == i2).astype(jnp.float32)
    return oh1, oh2, m1, m2


def _router_body(logits_ref, lt_ref, mtri_ref,
                 pos0_ref, pos1_ref, g0_ref, g1_ref, counts_ref,
                 counts_acc, run_acc, off_acc):
    p = pl.program_id(0)
    i = pl.program_id(1)
    logits = logits_ref[...]
    oh1, oh2, m1, m2 = _top2(logits)
    s = oh1 + oh2

    @pl.when(jnp.logical_and(p == 0, i == 0))
    def _init():
        counts_acc[...] = jnp.zeros_like(counts_acc)

    @pl.when(p == 0)
    def _count():
        counts_acc[...] += jnp.sum(s, axis=0, keepdims=True)
        counts_ref[...] = jnp.broadcast_to(counts_acc[...], counts_ref.shape)

    @pl.when(jnp.logical_and(p == 1, i == 0))
    def _offsets():
        c = counts_acc[...]
        padded = jnp.floor((c + (BM - 1)) / BM) * BM
        off_acc[...] = jnp.dot(padded, mtri_ref[...],
                               preferred_element_type=jnp.float32)
        run_acc[...] = jnp.zeros_like(run_acc)

    @pl.when(p == 1)
    def _emit():
        c_blk = jnp.dot(lt_ref[...], s, preferred_element_type=jnp.float32)
        tot = c_blk + run_acc[...] + off_acc[...]
        run_acc[...] += jnp.sum(s, axis=0, keepdims=True)
        pos0 = jnp.sum(tot * oh1, axis=1, keepdims=True).astype(jnp.int32)
        pos1 = jnp.sum(tot * oh2, axis=1, keepdims=True).astype(jnp.int32)
        pos0_ref[...] = jnp.broadcast_to(pos0, pos0_ref.shape)
        pos1_ref[...] = jnp.broadcast_to(pos1, pos1_ref.shape)
        g1v = 1.0 / (1.0 + jnp.exp(m2 - m1))
        g0_ref[...] = jnp.broadcast_to(g1v, g0_ref.shape)
        g1_ref[...] = jnp.broadcast_to(1.0 - g1v, g1_ref.shape)


def _run_router(logits_pad, lt, mtri):
    nb = T // TBLK

    def tok_spec(dtype):
        return (pl.BlockSpec((TBLK, LANES),
                             lambda p, i: (jnp.where(p == 1, i, nb), 0)),
                jax.ShapeDtypeStruct((T + TBLK, LANES), dtype))

    specs = [tok_spec(jnp.int32), tok_spec(jnp.int32),
             tok_spec(jnp.float32), tok_spec(jnp.float32)]
    return pl.pallas_call(
        _router_body,
        grid=(2, nb),
        in_specs=[
            pl.BlockSpec((TBLK, LANES), lambda p, i: (i, 0)),
            pl.BlockSpec((TBLK, TBLK), lambda p, i: (0, 0)),
            pl.BlockSpec((LANES, LANES), lambda p, i: (0, 0)),
        ],
        out_specs=[s for s, _ in specs] + [
            pl.BlockSpec((8, LANES), lambda p, i: (0, 0))],
        out_shape=[o for _, o in specs] + [
            jax.ShapeDtypeStruct((8, LANES), jnp.float32)],
        scratch_shapes=[
            pltpu.VMEM((1, LANES), jnp.float32),
            pltpu.VMEM((1, LANES), jnp.float32),
            pltpu.VMEM((1, LANES), jnp.float32),
        ],
    )(logits_pad, lt, mtri)


@functools.lru_cache(maxsize=None)
def _make_dispatch():
    mesh = plsc.VectorSubcoreMesh(core_axis_name="c", subcore_axis_name="s",
                                  num_cores=NC, num_subcores=NS)

    @functools.partial(
        pl.kernel,
        mesh=mesh,
        out_type=[
            jax.ShapeDtypeStruct((N_PAD, D), jnp.float32),
            jax.ShapeDtypeStruct((N_PAD, LANES), jnp.float32),
        ],
        scratch_types=[
            pltpu.VMEM((TPW, D), jnp.float32),
            pltpu.VMEM((TPW, LANES), jnp.float32),
            pltpu.VMEM((TPW, LANES), jnp.float32),
            pltpu.VMEM((TPW,), jnp.int32),
            pltpu.VMEM((TPW,), jnp.int32),
            pltpu.SemaphoreType.DMA,
        ],
    )
    def _dispatch(x_hbm, pos0_hbm, pos1_hbm, g0_hbm, g1_hbm,
                  xpad_hbm, gsort_hbm,
                  rows_v, g0_v, g1_v, idx0_v, idx1_v, sem):
        wid = lax.axis_index("s") * NC + lax.axis_index("c")
        base = wid * TPW
        pltpu.sync_copy(x_hbm.at[pl.ds(base, TPW)], rows_v)
        pltpu.sync_copy(pos0_hbm.at[pl.ds(base, TPW)], idx0_v)
        pltpu.sync_copy(pos1_hbm.at[pl.ds(base, TPW)], idx1_v)
        pltpu.sync_copy(g0_hbm.at[pl.ds(base, TPW)], g0_v)
        pltpu.sync_copy(g1_hbm.at[pl.ds(base, TPW)], g1_v)
        c0 = pltpu.async_copy(rows_v, xpad_hbm.at[idx0_v], sem)
        c1 = pltpu.async_copy(rows_v, xpad_hbm.at[idx1_v], sem)
        c2 = pltpu.async_copy(g0_v, gsort_hbm.at[idx0_v], sem)
        c3 = pltpu.async_copy(g1_v, gsort_hbm.at[idx1_v], sem)
        c0.wait()
        c1.wait()
        c2.wait()
        c3.wait()

    return _dispatch


FH = F // 2


def _gmm_body(be_ref, bv_ref, x_ref, g_ref, w1_ref, w2_ref, o_ref, acc_ref):
    f = pl.program_id(0)
    m = pl.program_id(1)

    @pl.when(bv_ref[m] == 1)
    def _():
        x = x_ref[...].astype(jnp.bfloat16)
        h = jnp.dot(x, w1_ref[0].astype(jnp.bfloat16),
                    preferred_element_type=jnp.float32)
        h = jax.nn.gelu(h)
        part = jnp.dot(h.astype(jnp.bfloat16), w2_ref[0].astype(jnp.bfloat16),
                       preferred_element_type=jnp.float32)
        sl = pl.ds(pl.multiple_of(m * BM, BM), BM)

        @pl.when(f == 0)
        def _store():
            acc_ref[sl, :] = part.astype(jnp.bfloat16)

        @pl.when(f == 1)
        def _emit():
            o_ref[...] = (acc_ref[sl, :].astype(jnp.float32) + part) \
                * g_ref[:, 0:1]


def _run_gmm(block_expert, block_valid, xpad, gsort, w1, w2):
    spec = pltpu.PrefetchScalarGridSpec(
        num_scalar_prefetch=2,
        grid=(2, NBLK),
        in_specs=[
            pl.BlockSpec((BM, D), lambda f, m, be, bv: (m, 0)),
            pl.BlockSpec((BM, LANES),
                         lambda f, m, be, bv: (jnp.where(f == 1, m, 0), 0)),
            pl.BlockSpec((1, D, FH), lambda f, m, be, bv: (be[m], 0, f)),
            pl.BlockSpec((1, FH, D), lambda f, m, be, bv: (be[m], f, 0)),
        ],
        out_specs=pl.BlockSpec(
            (BM, D),
            lambda f, m, be, bv: (jnp.where(f == 1, m, NBLK), 0)),
        scratch_shapes=[pltpu.VMEM((N_PAD, D), jnp.bfloat16)],
    )
    out = pl.pallas_call(
        _gmm_body,
        grid_spec=spec,
        out_shape=jax.ShapeDtypeStruct(((NBLK + 1) * BM, D), jnp.float32),
    )(block_expert, block_valid, xpad, gsort, w1, w2)
    return out


def _shared_body(x_ref, w1_ref, w2_ref, o_ref, acc_ref):
    f = pl.program_id(0)
    i = pl.program_id(1)
    x = x_ref[...].astype(jnp.bfloat16)
    h = jax.nn.gelu(jnp.dot(x, w1_ref[...].astype(jnp.bfloat16),
                            preferred_element_type=jnp.float32))
    part = jnp.dot(h.astype(jnp.bfloat16), w2_ref[...].astype(jnp.bfloat16),
                   preferred_element_type=jnp.float32)
    sl = pl.ds(pl.multiple_of(i * TBLK, TBLK), TBLK)

    @pl.when(f == 0)
    def _store():
        acc_ref[sl, :] = part.astype(jnp.bfloat16)

    @pl.when(f == 1)
    def _emit():
        o_ref[...] = acc_ref[sl, :].astype(jnp.float32) + part


def _run_shared(x, w1, w2):
    nb = T // TBLK
    return pl.pallas_call(
        _shared_body,
        grid=(2, nb),
        in_specs=[
            pl.BlockSpec((TBLK, D), lambda f, i: (i, 0)),
            pl.BlockSpec((D, FH), lambda f, i: (0, f)),
            pl.BlockSpec((FH, D), lambda f, i: (f, 0)),
        ],
        out_specs=pl.BlockSpec(
            (TBLK, D), lambda f, i: (jnp.where(f == 1, i, nb), 0)),
        out_shape=jax.ShapeDtypeStruct((T + TBLK, D), jnp.float32),
        scratch_shapes=[pltpu.VMEM((T, D), jnp.bfloat16)],
    )(x, w1, w2)


@functools.lru_cache(maxsize=None)
def _make_combine():
    mesh = plsc.VectorSubcoreMesh(core_axis_name="c", subcore_axis_name="s",
                                  num_cores=NC, num_subcores=NS)

    @functools.partial(
        pl.kernel,
        mesh=mesh,
        out_type=jax.ShapeDtypeStruct((T, D), jnp.float32),
        scratch_types=[
            pltpu.VMEM((CHUNK, D), jnp.float32),
            pltpu.VMEM((CHUNK, D), jnp.float32),
            pltpu.VMEM((CHUNK, D), jnp.float32),
            pltpu.VMEM((CHUNK,), jnp.int32),
            pltpu.VMEM((CHUNK,), jnp.int32),
            pltpu.SemaphoreType.DMA,
        ],
    )
    def _combine(shared_hbm, osort_hbm, pos0_hbm, pos1_hbm, out_hbm,
                 r0_v, r1_v, sh_v, idx0_v, idx1_v, sem):
        wid = lax.axis_index("s") * NC + lax.axis_index("c")
        for chunk in range(TPW // CHUNK):
            base = wid * TPW + chunk * CHUNK
            pltpu.sync_copy(pos0_hbm.at[pl.ds(base, CHUNK)], idx0_v)
            pltpu.sync_copy(pos1_hbm.at[pl.ds(base, CHUNK)], idx1_v)
            a = pltpu.async_copy(osort_hbm.at[idx0_v], r0_v, sem)
            b = pltpu.async_copy(osort_hbm.at[idx1_v], r1_v, sem)
            pltpu.sync_copy(shared_hbm.at[pl.ds(base, CHUNK)], sh_v)
            a.wait()
            b.wait()

            def col(j, _):
                sl = pl.ds(pl.multiple_of(j * 16, 16), 16)
                for i in range(CHUNK):
                    sh_v[i, sl] = sh_v[i, sl] + r0_v[i, sl] + r1_v[i, sl]
                return 0

            lax.fori_loop(0, D // 16, col, 0)
            pltpu.sync_copy(sh_v, out_hbm.at[pl.ds(base, CHUNK)])

    return _combine


_LT = np.tril(np.ones((TBLK, TBLK), np.float32), k=-1)
_MTRI = np.triu(np.ones((LANES, LANES), np.float32), k=1)


def kernel(hidden_states, router_w, shared_w1, shared_w2, expert_w1, expert_w2):
    logits = hidden_states @ router_w
    logits_pad = jnp.pad(logits, ((0, 0), (0, LANES - E)),
                         constant_values=-1e30)

    lt = jnp.asarray(_LT)
    mtri = jnp.asarray(_MTRI)
    pos0w, pos1w, g0w, g1w, counts_w = _run_router(logits_pad, lt, mtri)
    pos0 = pos0w[:T, 0]
    pos1 = pos1w[:T, 0]

    counts = counts_w[0, :E].astype(jnp.int32)
    padded = ((counts + BM - 1) // BM) * BM
    incl = jnp.cumsum(padded)
    starts = jnp.arange(NBLK, dtype=jnp.int32) * BM
    block_expert = jnp.minimum(
        jnp.searchsorted(incl, starts, side="right").astype(jnp.int32), E - 1)
    block_valid = (starts < incl[E - 1]).astype(jnp.int32)

    xpad, gsort = _make_dispatch()(hidden_states, pos0, pos1, g0w, g1w)

    osort = _run_gmm(block_expert, block_valid, xpad, gsort,
                     expert_w1, expert_w2)

    shared = _run_shared(hidden_states, shared_w1, shared_w2)

    return _make_combine()(shared, osort, pos0, pos1)


# speedup vs baseline: 1.0767x; 1.0452x over previous
import functools

import jax
import jax.numpy as jnp
import numpy as np
from jax import lax
from jax.experimental import pallas as pl
from jax.experimental.pallas import tpu as pltpu
from jax.experimental.pallas import tpu_sc as plsc

D = 1024
F = 4096
E = 8
T = 2048
LANES = 128
BM = 256
NBLK = (T * 2) // BM + E
TBLK = 512
N_PAD = NBLK * BM

NC, NS = 2, 16
NW = NC * NS
TPW = T // NW
CHUNK = 32

_NEG = np.float32(-1e30)


def _top2(logits):
    lane = lax.broadcasted_iota(jnp.int32, logits.shape, 1)
    m1 = jnp.max(logits, axis=1, keepdims=True)
    i1 = jnp.min(jnp.where(logits == m1, lane, LANES), axis=1, keepdims=True)
    oh1 = (lane == i1).astype(jnp.float32)
    l2 = jnp.where(lane == i1, _NEG, logits)
    m2 = jnp.max(l2, axis=1, keepdims=True)
    i2 = jnp.min(jnp.where(l2 == m2, lane, LANES), axis=1, keepdims=True)
    oh2 = (lane == i2).astype(jnp.float32)
    return oh1, oh2, m1, m2


def _router_body(logits_ref, lt_ref, mtri_ref,
                 pos0_ref, pos1_ref, g0_ref, g1_ref, counts_ref,
                 counts_acc, run_acc, off_acc):
    p = pl.program_id(0)
    i = pl.program_id(1)
    logits = logits_ref[...]
    oh1, oh2, m1, m2 = _top2(logits)
    s = oh1 + oh2

    @pl.when(jnp.logical_and(p == 0, i == 0))
    def _init():
        counts_acc[...] = jnp.zeros_like(counts_acc)

    @pl.when(p == 0)
    def _count():
        counts_acc[...] += jnp.sum(s, axis=0, keepdims=True)
        counts_ref[...] = jnp.broadcast_to(counts_acc[...], counts_ref.shape)

    @pl.when(jnp.logical_and(p == 1, i == 0))
    def _offsets():
        c = counts_acc[...]
        padded = jnp.floor((c + (BM - 1)) / BM) * BM
        off_acc[...] = jnp.dot(padded, mtri_ref[...],
                               preferred_element_type=jnp.float32)
        run_acc[...] = jnp.zeros_like(run_acc)

    @pl.when(p == 1)
    def _emit():
        c_blk = jnp.dot(lt_ref[...], s, preferred_element_type=jnp.float32)
        tot = c_blk + run_acc[...] + off_acc[...]
        run_acc[...] += jnp.sum(s, axis=0, keepdims=True)
        pos0 = jnp.sum(tot * oh1, axis=1, keepdims=True).astype(jnp.int32)
        pos1 = jnp.sum(tot * oh2, axis=1, keepdims=True).astype(jnp.int32)
        pos0_ref[...] = jnp.broadcast_to(pos0, pos0_ref.shape)
        pos1_ref[...] = jnp.broadcast_to(pos1, pos1_ref.shape)
        g1v = 1.0 / (1.0 + jnp.exp(m2 - m1))
        g0_ref[...] = jnp.broadcast_to(g1v, g0_ref.shape)
        g1_ref[...] = jnp.broadcast_to(1.0 - g1v, g1_ref.shape)


def _run_router(logits_pad, lt, mtri):
    nb = T // TBLK

    def tok_spec(dtype):
        return (pl.BlockSpec((TBLK, LANES),
                             lambda p, i: (jnp.where(p == 1, i, nb), 0)),
                jax.ShapeDtypeStruct((T + TBLK, LANES), dtype))

    specs = [tok_spec(jnp.int32), tok_spec(jnp.int32),
             tok_spec(jnp.float32), tok_spec(jnp.float32)]
    return pl.pallas_call(
        _router_body,
        grid=(2, nb),
        in_specs=[
            pl.BlockSpec((TBLK, LANES), lambda p, i: (i, 0)),
            pl.BlockSpec((TBLK, TBLK), lambda p, i: (0, 0)),
            pl.BlockSpec((LANES, LANES), lambda p, i: (0, 0)),
        ],
        out_specs=[s for s, _ in specs] + [
            pl.BlockSpec((8, LANES), lambda p, i: (0, 0))],
        out_shape=[o for _, o in specs] + [
            jax.ShapeDtypeStruct((8, LANES), jnp.float32)],
        scratch_shapes=[
            pltpu.VMEM((1, LANES), jnp.float32),
            pltpu.VMEM((1, LANES), jnp.float32),
            pltpu.VMEM((1, LANES), jnp.float32),
        ],
    )(logits_pad, lt, mtri)


@functools.lru_cache(maxsize=None)
def _make_dispatch():
    mesh = plsc.VectorSubcoreMesh(core_axis_name="c", subcore_axis_name="s",
                                  num_cores=NC, num_subcores=NS)

    @functools.partial(
        pl.kernel,
        mesh=mesh,
        out_type=[
            jax.ShapeDtypeStruct((N_PAD, D), jnp.float32),
            jax.ShapeDtypeStruct((N_PAD, LANES), jnp.float32),
        ],
        scratch_types=[
            pltpu.VMEM((TPW, D), jnp.float32),
            pltpu.VMEM((TPW, LANES), jnp.float32),
            pltpu.VMEM((TPW, LANES), jnp.float32),
            pltpu.VMEM((TPW,), jnp.int32),
            pltpu.VMEM((TPW,), jnp.int32),
            pltpu.SemaphoreType.DMA,
        ],
    )
    def _dispatch(x_hbm, pos0_hbm, pos1_hbm, g0_hbm, g1_hbm,
                  xpad_hbm, gsort_hbm,
                  rows_v, g0_v, g1_v, idx0_v, idx1_v, sem):
        wid = lax.axis_index("s") * NC + lax.axis_index("c")
        base = wid * TPW
        pltpu.sync_copy(x_hbm.at[pl.ds(base, TPW)], rows_v)
        pltpu.sync_copy(pos0_hbm.at[pl.ds(base, TPW)], idx0_v)
        pltpu.sync_copy(pos1_hbm.at[pl.ds(base, TPW)], idx1_v)
        pltpu.sync_copy(g0_hbm.at[pl.ds(base, TPW)], g0_v)
        pltpu.sync_copy(g1_hbm.at[pl.ds(base, TPW)], g1_v)
        c0 = pltpu.async_copy(rows_v, xpad_hbm.at[idx0_v], sem)
        c1 = pltpu.async_copy(rows_v, xpad_hbm.at[idx1_v], sem)
        c2 = pltpu.async_copy(g0_v, gsort_hbm.at[idx0_v], sem)
        c3 = pltpu.async_copy(g1_v, gsort_hbm.at[idx1_v], sem)
        c0.wait()
        c1.wait()
        c2.wait()
        c3.wait()

    return _dispatch


FH = F // 2


def _gmm_body(be_ref, bv_ref, x_ref, g_ref, w1_ref, w2_ref, o_ref, acc_ref):
    f = pl.program_id(0)
    m = pl.program_id(1)

    @pl.when(bv_ref[m] == 1)
    def _():
        x = x_ref[...].astype(jnp.bfloat16)
        h = jnp.dot(x, w1_ref[0].astype(jnp.bfloat16),
                    preferred_element_type=jnp.float32)
        h = jax.nn.gelu(h)
        part = jnp.dot(h.astype(jnp.bfloat16), w2_ref[0].astype(jnp.bfloat16),
                       preferred_element_type=jnp.float32)
        sl = pl.ds(pl.multiple_of(m * BM, BM), BM)

        @pl.when(f == 0)
        def _store():
            acc_ref[sl, :] = part.astype(jnp.bfloat16)

        @pl.when(f == 1)
        def _emit():
            o_ref[...] = (acc_ref[sl, :].astype(jnp.float32) + part) \
                * g_ref[:, 0:1]


def _run_gmm(block_expert, block_valid, xpad, gsort, w1, w2):
    spec = pltpu.PrefetchScalarGridSpec(
        num_scalar_prefetch=2,
        grid=(2, NBLK),
        in_specs=[
            pl.BlockSpec((BM, D), lambda f, m, be, bv: (m, 0)),
            pl.BlockSpec((BM, LANES),
                         lambda f, m, be, bv: (jnp.where(f == 1, m, 0), 0)),
            pl.BlockSpec((1, D, FH), lambda f, m, be, bv: (be[m], 0, f)),
            pl.BlockSpec((1, FH, D), lambda f, m, be, bv: (be[m], f, 0)),
        ],
        out_specs=pl.BlockSpec(
            (BM, D),
            lambda f, m, be, bv: (jnp.where(f == 1, m, NBLK), 0)),
        scratch_shapes=[pltpu.VMEM((N_PAD, D), jnp.bfloat16)],
    )
    out = pl.pallas_call(
        _gmm_body,
        grid_spec=spec,
        out_shape=jax.ShapeDtypeStruct(((NBLK + 1) * BM, D), jnp.float32),
    )(block_expert, block_valid, xpad, gsort, w1, w2)
    return out


def _shared_body(x_ref, w1_ref, w2_ref, o_ref, acc_ref):
    f = pl.program_id(0)
    i = pl.program_id(1)
    x = x_ref[...].astype(jnp.bfloat16)
    h = jax.nn.gelu(jnp.dot(x, w1_ref[...].astype(jnp.bfloat16),
                            preferred_element_type=jnp.float32))
    part = jnp.dot(h.astype(jnp.bfloat16), w2_ref[...].astype(jnp.bfloat16),
                   preferred_element_type=jnp.float32)
    sl = pl.ds(pl.multiple_of(i * TBLK, TBLK), TBLK)

    @pl.when(f == 0)
    def _store():
        acc_ref[sl, :] = part.astype(jnp.bfloat16)

    @pl.when(f == 1)
    def _emit():
        o_ref[...] = acc_ref[sl, :].astype(jnp.float32) + part


def _run_shared(x, w1, w2):
    nb = T // TBLK
    return pl.pallas_call(
        _shared_body,
        grid=(2, nb),
        in_specs=[
            pl.BlockSpec((TBLK, D), lambda f, i: (i, 0)),
            pl.BlockSpec((D, FH), lambda f, i: (0, f)),
            pl.BlockSpec((FH, D), lambda f, i: (f, 0)),
        ],
        out_specs=pl.BlockSpec(
            (TBLK, D), lambda f, i: (jnp.where(f == 1, i, nb), 0)),
        out_shape=jax.ShapeDtypeStruct((T + TBLK, D), jnp.float32),
        scratch_shapes=[pltpu.VMEM((T, D), jnp.bfloat16)],
    )(x, w1, w2)


@functools.lru_cache(maxsize=None)
def _make_combine():
    mesh = plsc.VectorSubcoreMesh(core_axis_name="c", subcore_axis_name="s",
                                  num_cores=NC, num_subcores=NS)

    @functools.partial(
        pl.kernel,
        mesh=mesh,
        out_type=jax.ShapeDtypeStruct((T, D), jnp.float32),
        scratch_types=[
            pltpu.VMEM((CHUNK, D), jnp.float32),
            pltpu.VMEM((CHUNK, D), jnp.float32),
            pltpu.VMEM((CHUNK, D), jnp.float32),
            pltpu.VMEM((CHUNK,), jnp.int32),
            pltpu.VMEM((CHUNK,), jnp.int32),
            pltpu.SemaphoreType.DMA,
        ],
    )
    def _combine(shared_hbm, osort_hbm, pos0_hbm, pos1_hbm, out_hbm,
                 r0_v, r1_v, sh_v, idx0_v, idx1_v, sem):
        wid = lax.axis_index("s") * NC + lax.axis_index("c")
        for chunk in range(TPW // CHUNK):
            base = wid * TPW + chunk * CHUNK
            pltpu.sync_copy(pos0_hbm.at[pl.ds(base, CHUNK)], idx0_v)
            pltpu.sync_copy(pos1_hbm.at[pl.ds(base, CHUNK)], idx1_v)
            a = pltpu.async_copy(osort_hbm.at[idx0_v], r0_v, sem)
            b = pltpu.async_copy(osort_hbm.at[idx1_v], r1_v, sem)
            pltpu.sync_copy(shared_hbm.at[pl.ds(base, CHUNK)], sh_v)
            a.wait()
            b.wait()

            def col(j, _):
                sl = pl.ds(pl.multiple_of(j * 16, 16), 16)
                for i in range(CHUNK):
                    sh_v[i, sl] = sh_v[i, sl] + r0_v[i, sl] + r1_v[i, sl]
                return 0

            lax.fori_loop(0, D // 16, col, 0)
            pltpu.sync_copy(sh_v, out_hbm.at[pl.ds(base, CHUNK)])

    return _combine


_LT = np.tril(np.ones((TBLK, TBLK), np.float32), k=-1)
_MTRI = np.triu(np.ones((LANES, LANES), np.float32), k=1)


def kernel(hidden_states, router_w, shared_w1, shared_w2, expert_w1, expert_w2):
    logits = hidden_states @ router_w
    logits_pad = jnp.pad(logits, ((0, 0), (0, LANES - E)),
                         constant_values=-1e30)

    lt = jnp.asarray(_LT)
    mtri = jnp.asarray(_MTRI)
    pos0w, pos1w, g0w, g1w, counts_w = _run_router(logits_pad, lt, mtri)
    pos0 = pos0w[:T, 0]
    pos1 = pos1w[:T, 0]

    counts = counts_w[0, :E].astype(jnp.int32)
    padded = ((counts + BM - 1) // BM) * BM
    incl = jnp.cumsum(padded)
    starts = jnp.arange(NBLK, dtype=jnp.int32) * BM
    block_expert = jnp.minimum(
        jnp.searchsorted(incl, starts, side="right").astype(jnp.int32), E - 1)
    block_valid = (starts < incl[E - 1]).astype(jnp.int32)

    xpad, gsort = _make_dispatch()(hidden_states, pos0, pos1, g0w, g1w)

    osort = _run_gmm(block_expert, block_valid, xpad, gsort,
                     expert_w1, expert_w2)

    shared = _run_shared(hidden_states, shared_w1, shared_w2)

    return _make_combine()(shared, osort, pos0, pos1)


# speedup vs baseline: 1.0827x; 1.0056x over previous
import functools

import jax
import jax.numpy as jnp
import numpy as np
from jax import lax
from jax.experimental import pallas as pl
from jax.experimental.pallas import tpu as pltpu
from jax.experimental.pallas import tpu_sc as plsc

D = 1024
F = 4096
E = 8
T = 2048
LANES = 128
BM = 256
NBLK = (T * 2) // BM + E
TBLK = 1024
N_PAD = NBLK * BM

NC, NS = 2, 16
NW = NC * NS
TPW = T // NW
CHUNK = 32

_NEG = np.float32(-1e30)


def _top2(logits):
    lane = lax.broadcasted_iota(jnp.int32, logits.shape, 1)
    m1 = jnp.max(logits, axis=1, keepdims=True)
    i1 = jnp.min(jnp.where(logits == m1, lane, LANES), axis=1, keepdims=True)
    oh1 = (lane == i1).astype(jnp.float32)
    l2 = jnp.where(lane == i1, _NEG, logits)
    m2 = jnp.max(l2, axis=1, keepdims=True)
    i2 = jnp.min(jnp.where(l2 == m2, lane, LANES), axis=1, keepdims=True)
    oh2 = (lane == i2).astype(jnp.float32)
    return oh1, oh2, m1, m2


def _router_body(logits_ref, lt_ref, mtri_ref,
                 pos0_ref, pos1_ref, g0_ref, g1_ref, counts_ref,
                 counts_acc, run_acc, off_acc):
    p = pl.program_id(0)
    i = pl.program_id(1)
    logits = logits_ref[...]
    oh1, oh2, m1, m2 = _top2(logits)
    s = oh1 + oh2

    @pl.when(jnp.logical_and(p == 0, i == 0))
    def _init():
        counts_acc[...] = jnp.zeros_like(counts_acc)

    @pl.when(p == 0)
    def _count():
        counts_acc[...] += jnp.sum(s, axis=0, keepdims=True)
        counts_ref[...] = jnp.broadcast_to(counts_acc[...], counts_ref.shape)

    @pl.when(jnp.logical_and(p == 1, i == 0))
    def _offsets():
        c = counts_acc[...]
        padded = jnp.floor((c + (BM - 1)) / BM) * BM
        off_acc[...] = jnp.dot(padded, mtri_ref[...],
                               preferred_element_type=jnp.float32)
        run_acc[...] = jnp.zeros_like(run_acc)

    @pl.when(p == 1)
    def _emit():
        c_blk = jnp.dot(lt_ref[...], s, preferred_element_type=jnp.float32)
        tot = c_blk + run_acc[...] + off_acc[...]
        run_acc[...] += jnp.sum(s, axis=0, keepdims=True)
        pos0 = jnp.sum(tot * oh1, axis=1, keepdims=True).astype(jnp.int32)
        pos1 = jnp.sum(tot * oh2, axis=1, keepdims=True).astype(jnp.int32)
        pos0_ref[...] = jnp.broadcast_to(pos0, pos0_ref.shape)
        pos1_ref[...] = jnp.broadcast_to(pos1, pos1_ref.shape)
        g1v = 1.0 / (1.0 + jnp.exp(m2 - m1))
        g0_ref[...] = jnp.broadcast_to(g1v, g0_ref.shape)
        g1_ref[...] = jnp.broadcast_to(1.0 - g1v, g1_ref.shape)


def _run_router(logits_pad, lt, mtri):
    nb = T // TBLK

    def tok_spec(dtype):
        return (pl.BlockSpec((TBLK, LANES),
                             lambda p, i: (jnp.where(p == 1, i, nb), 0)),
                jax.ShapeDtypeStruct((T + TBLK, LANES), dtype))

    specs = [tok_spec(jnp.int32), tok_spec(jnp.int32),
             tok_spec(jnp.float32), tok_spec(jnp.float32)]
    return pl.pallas_call(
        _router_body,
        grid=(2, nb),
        in_specs=[
            pl.BlockSpec((TBLK, LANES), lambda p, i: (i, 0)),
            pl.BlockSpec((TBLK, TBLK), lambda p, i: (0, 0)),
            pl.BlockSpec((LANES, LANES), lambda p, i: (0, 0)),
        ],
        out_specs=[s for s, _ in specs] + [
            pl.BlockSpec((8, LANES), lambda p, i: (0, 0))],
        out_shape=[o for _, o in specs] + [
            jax.ShapeDtypeStruct((8, LANES), jnp.float32)],
        scratch_shapes=[
            pltpu.VMEM((1, LANES), jnp.float32),
            pltpu.VMEM((1, LANES), jnp.float32),
            pltpu.VMEM((1, LANES), jnp.float32),
        ],
    )(logits_pad, lt, mtri)


@functools.lru_cache(maxsize=None)
def _make_dispatch():
    mesh = plsc.VectorSubcoreMesh(core_axis_name="c", subcore_axis_name="s",
                                  num_cores=NC, num_subcores=NS)

    @functools.partial(
        pl.kernel,
        mesh=mesh,
        out_type=[
            jax.ShapeDtypeStruct((N_PAD, D), jnp.float32),
            jax.ShapeDtypeStruct((N_PAD, LANES), jnp.float32),
        ],
        scratch_types=[
            pltpu.VMEM((TPW, D), jnp.float32),
            pltpu.VMEM((TPW, LANES), jnp.float32),
            pltpu.VMEM((TPW, LANES), jnp.float32),
            pltpu.VMEM((TPW,), jnp.int32),
            pltpu.VMEM((TPW,), jnp.int32),
            pltpu.SemaphoreType.DMA,
        ],
    )
    def _dispatch(x_hbm, pos0_hbm, pos1_hbm, g0_hbm, g1_hbm,
                  xpad_hbm, gsort_hbm,
                  rows_v, g0_v, g1_v, idx0_v, idx1_v, sem):
        wid = lax.axis_index("s") * NC + lax.axis_index("c")
        base = wid * TPW
        pltpu.sync_copy(x_hbm.at[pl.ds(base, TPW)], rows_v)
        pltpu.sync_copy(pos0_hbm.at[pl.ds(base, TPW)], idx0_v)
        pltpu.sync_copy(pos1_hbm.at[pl.ds(base, TPW)], idx1_v)
        pltpu.sync_copy(g0_hbm.at[pl.ds(base, TPW)], g0_v)
        pltpu.sync_copy(g1_hbm.at[pl.ds(base, TPW)], g1_v)
        c0 = pltpu.async_copy(rows_v, xpad_hbm.at[idx0_v], sem)
        c1 = pltpu.async_copy(rows_v, xpad_hbm.at[idx1_v], sem)
        c2 = pltpu.async_copy(g0_v, gsort_hbm.at[idx0_v], sem)
        c3 = pltpu.async_copy(g1_v, gsort_hbm.at[idx1_v], sem)
        c0.wait()
        c1.wait()
        c2.wait()
        c3.wait()

    return _dispatch


FH = F // 2


def _gmm_body(be_ref, bv_ref, x_ref, g_ref, w1_ref, w2_ref, o_ref, acc_ref):
    f = pl.program_id(0)
    m = pl.program_id(1)

    @pl.when(bv_ref[m] == 1)
    def _():
        x = x_ref[...].astype(jnp.bfloat16)
        h = jnp.dot(x, w1_ref[0].astype(jnp.bfloat16),
                    preferred_element_type=jnp.float32)
        h = jax.nn.gelu(h)
        part = jnp.dot(h.astype(jnp.bfloat16), w2_ref[0].astype(jnp.bfloat16),
                       preferred_element_type=jnp.float32)
        sl = pl.ds(pl.multiple_of(m * BM, BM), BM)

        @pl.when(f == 0)
        def _store():
            acc_ref[sl, :] = part.astype(jnp.bfloat16)

        @pl.when(f == 1)
        def _emit():
            o_ref[...] = (acc_ref[sl, :].astype(jnp.float32) + part) \
                * g_ref[:, 0:1]


def _run_gmm(block_expert, block_valid, xpad, gsort, w1, w2):
    spec = pltpu.PrefetchScalarGridSpec(
        num_scalar_prefetch=2,
        grid=(2, NBLK),
        in_specs=[
            pl.BlockSpec((BM, D), lambda f, m, be, bv: (m, 0)),
            pl.BlockSpec((BM, LANES),
                         lambda f, m, be, bv: (jnp.where(f == 1, m, 0), 0)),
            pl.BlockSpec((1, D, FH), lambda f, m, be, bv: (be[m], 0, f)),
            pl.BlockSpec((1, FH, D), lambda f, m, be, bv: (be[m], f, 0)),
        ],
        out_specs=pl.BlockSpec(
            (BM, D),
            lambda f, m, be, bv: (jnp.where(f == 1, m, NBLK), 0)),
        scratch_shapes=[pltpu.VMEM((N_PAD, D), jnp.bfloat16)],
    )
    out = pl.pallas_call(
        _gmm_body,
        grid_spec=spec,
        out_shape=jax.ShapeDtypeStruct(((NBLK + 1) * BM, D), jnp.float32),
    )(block_expert, block_valid, xpad, gsort, w1, w2)
    return out


def _shared_body(x_ref, w1_ref, w2_ref, o_ref, acc_ref):
    f = pl.program_id(0)
    i = pl.program_id(1)
    x = x_ref[...].astype(jnp.bfloat16)
    h = jax.nn.gelu(jnp.dot(x, w1_ref[...].astype(jnp.bfloat16),
                            preferred_element_type=jnp.float32))
    part = jnp.dot(h.astype(jnp.bfloat16), w2_ref[...].astype(jnp.bfloat16),
                   preferred_element_type=jnp.float32)
    sl = pl.ds(pl.multiple_of(i * TBLK, TBLK), TBLK)

    @pl.when(f == 0)
    def _store():
        acc_ref[sl, :] = part.astype(jnp.bfloat16)

    @pl.when(f == 1)
    def _emit():
        o_ref[...] = acc_ref[sl, :].astype(jnp.float32) + part


def _run_shared(x, w1, w2):
    nb = T // TBLK
    return pl.pallas_call(
        _shared_body,
        grid=(2, nb),
        in_specs=[
            pl.BlockSpec((TBLK, D), lambda f, i: (i, 0)),
            pl.BlockSpec((D, FH), lambda f, i: (0, f)),
            pl.BlockSpec((FH, D), lambda f, i: (f, 0)),
        ],
        out_specs=pl.BlockSpec(
            (TBLK, D), lambda f, i: (jnp.where(f == 1, i, nb), 0)),
        out_shape=jax.ShapeDtypeStruct((T + TBLK, D), jnp.float32),
        scratch_shapes=[pltpu.VMEM((T, D), jnp.bfloat16)],
    )(x, w1, w2)


@functools.lru_cache(maxsize=None)
def _make_combine():
    mesh = plsc.VectorSubcoreMesh(core_axis_name="c", subcore_axis_name="s",
                                  num_cores=NC, num_subcores=NS)

    @functools.partial(
        pl.kernel,
        mesh=mesh,
        out_type=jax.ShapeDtypeStruct((T, D), jnp.float32),
        scratch_types=[
            pltpu.VMEM((CHUNK, D), jnp.float32),
            pltpu.VMEM((CHUNK, D), jnp.float32),
            pltpu.VMEM((CHUNK, D), jnp.float32),
            pltpu.VMEM((CHUNK,), jnp.int32),
            pltpu.VMEM((CHUNK,), jnp.int32),
            pltpu.SemaphoreType.DMA,
        ],
    )
    def _combine(shared_hbm, osort_hbm, pos0_hbm, pos1_hbm, out_hbm,
                 r0_v, r1_v, sh_v, idx0_v, idx1_v, sem):
        wid = lax.axis_index("s") * NC + lax.axis_index("c")
        for chunk in range(TPW // CHUNK):
            base = wid * TPW + chunk * CHUNK
            pltpu.sync_copy(pos0_hbm.at[pl.ds(base, CHUNK)], idx0_v)
            pltpu.sync_copy(pos1_hbm.at[pl.ds(base, CHUNK)], idx1_v)
            a = pltpu.async_copy(osort_hbm.at[idx0_v], r0_v, sem)
            b = pltpu.async_copy(osort_hbm.at[idx1_v], r1_v, sem)
            pltpu.sync_copy(shared_hbm.at[pl.ds(base, CHUNK)], sh_v)
            a.wait()
            b.wait()

            def col(j, _):
                sl = pl.ds(pl.multiple_of(j * 16, 16), 16)
                for i in range(CHUNK):
                    sh_v[i, sl] = sh_v[i, sl] + r0_v[i, sl] + r1_v[i, sl]
                return 0

            lax.fori_loop(0, D // 16, col, 0)
            pltpu.sync_copy(sh_v, out_hbm.at[pl.ds(base, CHUNK)])

    return _combine


_LT = np.tril(np.ones((TBLK, TBLK), np.float32), k=-1)
_MTRI = np.triu(np.ones((LANES, LANES), np.float32), k=1)


def kernel(hidden_states, router_w, shared_w1, shared_w2, expert_w1, expert_w2):
    logits = hidden_states @ router_w
    logits_pad = jnp.pad(logits, ((0, 0), (0, LANES - E)),
                         constant_values=-1e30)

    lt = jnp.asarray(_LT)
    mtri = jnp.asarray(_MTRI)
    pos0w, pos1w, g0w, g1w, counts_w = _run_router(logits_pad, lt, mtri)
    pos0 = pos0w[:T, 0]
    pos1 = pos1w[:T, 0]

    counts = counts_w[0, :E].astype(jnp.int32)
    padded = ((counts + BM - 1) // BM) * BM
    incl = jnp.cumsum(padded)
    starts = jnp.arange(NBLK, dtype=jnp.int32) * BM
    block_expert = jnp.minimum(
        jnp.searchsorted(incl, starts, side="right").astype(jnp.int32), E - 1)
    block_valid = (starts < incl[E - 1]).astype(jnp.int32)

    xpad, gsort = _make_dispatch()(hidden_states, pos0, pos1, g0w, g1w)

    osort = _run_gmm(block_expert, block_valid, xpad, gsort,
                     expert_w1, expert_w2)

    shared = _run_shared(hidden_states, shared_w1, shared_w2)

    return _make_combine()(shared, osort, pos0, pos1)
